# Initial kernel scaffold; baseline (speedup 1.0000x reference)
import jax
import jax.numpy as jnp
from jax.experimental import pallas as pl


def kernel(x, W0a, g0a, b0a, W0b, g0b, b0b, W1a, g1a, b1a, W1b, g1b, b1b, W2, g2, b2, W3, g3, b3, Wt, gt, bt):
    raise NotImplementedError("write your pallas kernel here")


# speedup vs baseline: 3.0800x; 3.0800x over previous
import jax
import jax.numpy as jnp
from jax.experimental import pallas as pl

_K = 20
_R = 256

_HI = jax.lax.Precision.HIGHEST


def _dot(a, b, dims):
    return jax.lax.dot_general(a, b, (dims, ((), ())), precision=_HI,
                               preferred_element_type=jnp.float32)


def _fused_edge_gather(xt, Wn, Wvc):
    B, N, C = xt.shape
    O = Wn.shape[0]

    def kern(xa_ref, xr_ref, wn_ref, wv_ref, y_ref, s_ref, ss_ref):
        b = pl.program_id(0)
        i = pl.program_id(1)
        xa = xa_ref[0]
        xr = xr_ref[0]
        u = _dot(xa, wn_ref[...], ((1,), (1,)))
        v = _dot(xr, wv_ref[...], ((1,), (1,)))
        g = _dot(xr, xa, ((1,), (1,)))
        xxr = jnp.sum(xr * xr, axis=1)
        xxa = jnp.sum(xa * xa, axis=1)
        p = 2.0 * g - xxr[:, None] - xxa[None, :]
        cols = jax.lax.broadcasted_iota(jnp.int32, (_R, N), 1)
        s = jnp.zeros((O,), jnp.float32)
        ss = jnp.zeros((O,), jnp.float32)
        for k in range(_K):
            m = jnp.max(p, axis=1)
            am = jnp.min(jnp.where(p == m[:, None], cols, N), axis=1)
            sel = cols == am[:, None]
            oh = sel.astype(jnp.float32)
            yk = _dot(oh, u, ((1,), (0,))) + v
            y_ref[0, :, k, :] = yk
            s = s + jnp.sum(yk, axis=0)
            ss = ss + jnp.sum(yk * yk, axis=0)
            p = jnp.where(sel, -jnp.inf, p)

        @pl.when((b == 0) & (i == 0))
        def _():
            s_ref[...] = jnp.zeros_like(s_ref)
            ss_ref[...] = jnp.zeros_like(ss_ref)

        s_ref[...] += s[None, :]
        ss_ref[...] += ss[None, :]

    return pl.pallas_call(
        kern,
        grid=(B, N // _R),
        in_specs=[
            pl.BlockSpec((1, N, C), lambda b, i: (b, 0, 0)),
            pl.BlockSpec((1, _R, C), lambda b, i: (b, i, 0)),
            pl.BlockSpec((O, C), lambda b, i: (0, 0)),
            pl.BlockSpec((O, C), lambda b, i: (0, 0)),
        ],
        out_specs=[
            pl.BlockSpec((1, _R, _K, O), lambda b, i: (b, i, 0, 0)),
            pl.BlockSpec((1, O), lambda b, i: (0, 0)),
            pl.BlockSpec((1, O), lambda b, i: (0, 0)),
        ],
        out_shape=[
            jax.ShapeDtypeStruct((B, N, _K, O), jnp.float32),
            jax.ShapeDtypeStruct((1, O), jnp.float32),
            jax.ShapeDtypeStruct((1, O), jnp.float32),
        ],
    )(xt, xt, Wn, Wvc)


def _bn_affine(s, ss, count, gamma, beta):
    mean = s[0] / count
    var = ss[0] / count - mean * mean
    sc = gamma / jnp.sqrt(var + 1e-5)
    sh = beta - mean * sc
    return sc[None, :], sh[None, :]


def _edge_mm2(y1, sc, sh, Wb):
    B, N, Kk, O = y1.shape
    O2 = Wb.shape[0]

    def kern(y_ref, sc_ref, sh_ref, w_ref, o_ref, s_ref, ss_ref):
        b = pl.program_id(0)
        i = pl.program_id(1)
        z = y_ref[0] * sc_ref[0][None, None, :] + sh_ref[0][None, None, :]
        z = jnp.where(z > 0, z, 0.2 * z)
        y2 = _dot(z.reshape(_R * Kk, O), w_ref[...], ((1,), (1,)))
        o_ref[0] = y2.reshape(_R, Kk, O2)

        @pl.when((b == 0) & (i == 0))
        def _():
            s_ref[...] = jnp.zeros_like(s_ref)
            ss_ref[...] = jnp.zeros_like(ss_ref)

        s_ref[...] += jnp.sum(y2, axis=0)[None, :]
        ss_ref[...] += jnp.sum(y2 * y2, axis=0)[None, :]

    return pl.pallas_call(
        kern,
        grid=(B, N // _R),
        in_specs=[
            pl.BlockSpec((1, _R, Kk, O), lambda b, i: (b, i, 0, 0)),
            pl.BlockSpec((1, O), lambda b, i: (0, 0)),
            pl.BlockSpec((1, O), lambda b, i: (0, 0)),
            pl.BlockSpec((O2, O), lambda b, i: (0, 0)),
        ],
        out_specs=[
            pl.BlockSpec((1, _R, Kk, O2), lambda b, i: (b, i, 0, 0)),
            pl.BlockSpec((1, O2), lambda b, i: (0, 0)),
            pl.BlockSpec((1, O2), lambda b, i: (0, 0)),
        ],
        out_shape=[
            jax.ShapeDtypeStruct((B, N, Kk, O2), jnp.float32),
            jax.ShapeDtypeStruct((1, O2), jnp.float32),
            jax.ShapeDtypeStruct((1, O2), jnp.float32),
        ],
    )(y1, sc, sh, Wb)


def _bn_max(y2, sc, sh):
    B, N, Kk, O = y2.shape

    def kern(y_ref, sc_ref, sh_ref, o_ref):
        z = y_ref[0] * sc_ref[0][None, None, :] + sh_ref[0][None, None, :]
        z = jnp.where(z > 0, z, 0.2 * z)
        o_ref[0] = jnp.max(z, axis=1)

    return pl.pallas_call(
        kern,
        grid=(B, N // _R),
        in_specs=[
            pl.BlockSpec((1, _R, Kk, O), lambda b, i: (b, i, 0, 0)),
            pl.BlockSpec((1, O), lambda b, i: (0, 0)),
            pl.BlockSpec((1, O), lambda b, i: (0, 0)),
        ],
        out_specs=pl.BlockSpec((1, _R, O), lambda b, i: (b, i, 0)),
        out_shape=jax.ShapeDtypeStruct((B, N, O), jnp.float32),
    )(y2, sc, sh)


def _final_mm(x1, x2, x3, x4, Wt):
    B, N, O = x1.shape
    O2 = Wt.shape[0]
    W1 = Wt[:, 0 * O:1 * O]
    W2 = Wt[:, 1 * O:2 * O]
    W3 = Wt[:, 2 * O:3 * O]
    W4 = Wt[:, 3 * O:4 * O]

    def kern(a_ref, b_ref, c_ref, d_ref, w1_ref, w2_ref, w3_ref, w4_ref,
             o_ref, s_ref, ss_ref):
        b = pl.program_id(0)
        i = pl.program_id(1)
        y = (_dot(a_ref[0], w1_ref[...], ((1,), (1,))) +
             _dot(b_ref[0], w2_ref[...], ((1,), (1,))) +
             _dot(c_ref[0], w3_ref[...], ((1,), (1,))) +
             _dot(d_ref[0], w4_ref[...], ((1,), (1,))))
        o_ref[0] = y

        @pl.when((b == 0) & (i == 0))
        def _():
            s_ref[...] = jnp.zeros_like(s_ref)
            ss_ref[...] = jnp.zeros_like(ss_ref)

        s_ref[...] += jnp.sum(y, axis=0)[None, :]
        ss_ref[...] += jnp.sum(y * y, axis=0)[None, :]

    wspec = pl.BlockSpec((O2, O), lambda b, i: (0, 0))
    xspec = pl.BlockSpec((1, _R, O), lambda b, i: (b, i, 0))
    return pl.pallas_call(
        kern,
        grid=(B, N // _R),
        in_specs=[xspec, xspec, xspec, xspec, wspec, wspec, wspec, wspec],
        out_specs=[
            pl.BlockSpec((1, _R, O2), lambda b, i: (b, i, 0)),
            pl.BlockSpec((1, O2), lambda b, i: (0, 0)),
            pl.BlockSpec((1, O2), lambda b, i: (0, 0)),
        ],
        out_shape=[
            jax.ShapeDtypeStruct((B, N, O2), jnp.float32),
            jax.ShapeDtypeStruct((1, O2), jnp.float32),
            jax.ShapeDtypeStruct((1, O2), jnp.float32),
        ],
    )(x1, x2, x3, x4, W1, W2, W3, W4)


def _final_bn(ypre, sc, sh):
    B, N, O = ypre.shape

    def kern(y_ref, sc_ref, sh_ref, o_ref):
        z = y_ref[0] * sc_ref[0][None, :] + sh_ref[0][None, :]
        z = jnp.where(z > 0, z, 0.2 * z)
        o_ref[0] = z.T

    return pl.pallas_call(
        kern,
        grid=(B, N // _R),
        in_specs=[
            pl.BlockSpec((1, _R, O), lambda b, i: (b, i, 0)),
            pl.BlockSpec((1, O), lambda b, i: (0, 0)),
            pl.BlockSpec((1, O), lambda b, i: (0, 0)),
        ],
        out_specs=pl.BlockSpec((1, O, _R), lambda b, i: (b, 0, i)),
        out_shape=jax.ShapeDtypeStruct((B, O, N), jnp.float32),
    )(ypre, sc, sh)


def _edge_stage(xt, Wa, ga, ba, Wb=None, gb=None, bb=None):
    B, N, C = xt.shape
    Wn = Wa[:, :C]
    Wvc = Wa[:, C:] - Wa[:, :C]
    y1, s1, ss1 = _fused_edge_gather(xt, Wn, Wvc)
    cnt = float(B * N * _K)
    sc1, sh1 = _bn_affine(s1, ss1, cnt, ga, ba)
    if Wb is None:
        return _bn_max(y1, sc1, sh1)
    y2, s2, ss2 = _edge_mm2(y1, sc1, sh1, Wb)
    sc2, sh2 = _bn_affine(s2, ss2, cnt, gb, bb)
    return _bn_max(y2, sc2, sh2)


def kernel(x, W0a, g0a, b0a, W0b, g0b, b0b, W1a, g1a, b1a, W1b, g1b, b1b,
           W2, g2, b2, W3, g3, b3, Wt, gt, bt):
    xt = jnp.transpose(x, (0, 2, 1))
    x1 = _edge_stage(xt, W0a, g0a, b0a, W0b, g0b, b0b)
    x2 = _edge_stage(x1, W1a, g1a, b1a, W1b, g1b, b1b)
    x3 = _edge_stage(x2, W2, g2, b2)
    x4 = _edge_stage(x3, W3, g3, b3)
    ypre, st, sst = _final_mm(x1, x2, x3, x4, Wt)
    B, N, _ = x1.shape
    sct, sht = _bn_affine(st, sst, float(B * N), gt, bt)
    return _final_bn(ypre, sct, sht)


# speedup vs baseline: 3.1547x; 1.0243x over previous
import jax
import jax.numpy as jnp
from jax.experimental import pallas as pl

_K = 20
_R = 256

_HI = jax.lax.Precision.HIGHEST


def _dot(a, b, dims):
    return jax.lax.dot_general(a, b, (dims, ((), ())), precision=_HI,
                               preferred_element_type=jnp.float32)


def _fused_edge_gather(xt, Wn, Wvc):
    B, N, C = xt.shape
    O = Wn.shape[0]

    def kern(xa_ref, xr_ref, wn_ref, wv_ref, y_ref, s_ref, ss_ref):
        b = pl.program_id(0)
        i = pl.program_id(1)
        xa = xa_ref[0]
        xr = xr_ref[0]
        u = _dot(xa, wn_ref[...], ((1,), (1,)))
        v = _dot(xr, wv_ref[...], ((1,), (1,)))
        g = _dot(xr, xa, ((1,), (1,)))
        xxr = jnp.sum(xr * xr, axis=1)
        xxa = jnp.sum(xa * xa, axis=1)
        p = 2.0 * g - xxr[:, None] - xxa[None, :]
        cols = jax.lax.broadcasted_iota(jnp.int32, (_R, N), 1)
        s = jnp.zeros((O,), jnp.float32)
        ss = jnp.zeros((O,), jnp.float32)
        for k in range(_K):
            am = jnp.argmax(p, axis=1).astype(jnp.int32)
            oh = (cols == am[:, None]).astype(jnp.float32)
            yk = _dot(oh, u, ((1,), (0,))) + v
            y_ref[0, :, k, :] = yk
            s = s + jnp.sum(yk, axis=0)
            ss = ss + jnp.sum(yk * yk, axis=0)
            p = p - oh * 1e30

        @pl.when((b == 0) & (i == 0))
        def _():
            s_ref[...] = jnp.zeros_like(s_ref)
            ss_ref[...] = jnp.zeros_like(ss_ref)

        s_ref[...] += s[None, :]
        ss_ref[...] += ss[None, :]

    return pl.pallas_call(
        kern,
        grid=(B, N // _R),
        in_specs=[
            pl.BlockSpec((1, N, C), lambda b, i: (b, 0, 0)),
            pl.BlockSpec((1, _R, C), lambda b, i: (b, i, 0)),
            pl.BlockSpec((O, C), lambda b, i: (0, 0)),
            pl.BlockSpec((O, C), lambda b, i: (0, 0)),
        ],
        out_specs=[
            pl.BlockSpec((1, _R, _K, O), lambda b, i: (b, i, 0, 0)),
            pl.BlockSpec((1, O), lambda b, i: (0, 0)),
            pl.BlockSpec((1, O), lambda b, i: (0, 0)),
        ],
        out_shape=[
            jax.ShapeDtypeStruct((B, N, _K, O), jnp.float32),
            jax.ShapeDtypeStruct((1, O), jnp.float32),
            jax.ShapeDtypeStruct((1, O), jnp.float32),
        ],
    )(xt, xt, Wn, Wvc)


def _bn_affine(s, ss, count, gamma, beta):
    mean = s[0] / count
    var = ss[0] / count - mean * mean
    sc = gamma / jnp.sqrt(var + 1e-5)
    sh = beta - mean * sc
    return sc[None, :], sh[None, :]


def _edge_mm2(y1, sc, sh, Wb):
    B, N, Kk, O = y1.shape
    O2 = Wb.shape[0]

    def kern(y_ref, sc_ref, sh_ref, w_ref, o_ref, s_ref, ss_ref):
        b = pl.program_id(0)
        i = pl.program_id(1)
        z = y_ref[0] * sc_ref[0][None, None, :] + sh_ref[0][None, None, :]
        z = jnp.where(z > 0, z, 0.2 * z)
        y2 = _dot(z.reshape(_R * Kk, O), w_ref[...], ((1,), (1,)))
        o_ref[0] = y2.reshape(_R, Kk, O2)

        @pl.when((b == 0) & (i == 0))
        def _():
            s_ref[...] = jnp.zeros_like(s_ref)
            ss_ref[...] = jnp.zeros_like(ss_ref)

        s_ref[...] += jnp.sum(y2, axis=0)[None, :]
        ss_ref[...] += jnp.sum(y2 * y2, axis=0)[None, :]

    return pl.pallas_call(
        kern,
        grid=(B, N // _R),
        in_specs=[
            pl.BlockSpec((1, _R, Kk, O), lambda b, i: (b, i, 0, 0)),
            pl.BlockSpec((1, O), lambda b, i: (0, 0)),
            pl.BlockSpec((1, O), lambda b, i: (0, 0)),
            pl.BlockSpec((O2, O), lambda b, i: (0, 0)),
        ],
        out_specs=[
            pl.BlockSpec((1, _R, Kk, O2), lambda b, i: (b, i, 0, 0)),
            pl.BlockSpec((1, O2), lambda b, i: (0, 0)),
            pl.BlockSpec((1, O2), lambda b, i: (0, 0)),
        ],
        out_shape=[
            jax.ShapeDtypeStruct((B, N, Kk, O2), jnp.float32),
            jax.ShapeDtypeStruct((1, O2), jnp.float32),
            jax.ShapeDtypeStruct((1, O2), jnp.float32),
        ],
    )(y1, sc, sh, Wb)


def _bn_max(y2, sc, sh):
    B, N, Kk, O = y2.shape

    def kern(y_ref, sc_ref, sh_ref, o_ref):
        z = y_ref[0] * sc_ref[0][None, None, :] + sh_ref[0][None, None, :]
        z = jnp.where(z > 0, z, 0.2 * z)
        o_ref[0] = jnp.max(z, axis=1)

    return pl.pallas_call(
        kern,
        grid=(B, N // _R),
        in_specs=[
            pl.BlockSpec((1, _R, Kk, O), lambda b, i: (b, i, 0, 0)),
            pl.BlockSpec((1, O), lambda b, i: (0, 0)),
            pl.BlockSpec((1, O), lambda b, i: (0, 0)),
        ],
        out_specs=pl.BlockSpec((1, _R, O), lambda b, i: (b, i, 0)),
        out_shape=jax.ShapeDtypeStruct((B, N, O), jnp.float32),
    )(y2, sc, sh)


def _final_mm(x1, x2, x3, x4, Wt):
    B, N, O = x1.shape
    O2 = Wt.shape[0]
    W1 = Wt[:, 0 * O:1 * O]
    W2 = Wt[:, 1 * O:2 * O]
    W3 = Wt[:, 2 * O:3 * O]
    W4 = Wt[:, 3 * O:4 * O]

    def kern(a_ref, b_ref, c_ref, d_ref, w1_ref, w2_ref, w3_ref, w4_ref,
             o_ref, s_ref, ss_ref):
        b = pl.program_id(0)
        i = pl.program_id(1)
        y = (_dot(a_ref[0], w1_ref[...], ((1,), (1,))) +
             _dot(b_ref[0], w2_ref[...], ((1,), (1,))) +
             _dot(c_ref[0], w3_ref[...], ((1,), (1,))) +
             _dot(d_ref[0], w4_ref[...], ((1,), (1,))))
        o_ref[0] = y

        @pl.when((b == 0) & (i == 0))
        def _():
            s_ref[...] = jnp.zeros_like(s_ref)
            ss_ref[...] = jnp.zeros_like(ss_ref)

        s_ref[...] += jnp.sum(y, axis=0)[None, :]
        ss_ref[...] += jnp.sum(y * y, axis=0)[None, :]

    wspec = pl.BlockSpec((O2, O), lambda b, i: (0, 0))
    xspec = pl.BlockSpec((1, _R, O), lambda b, i: (b, i, 0))
    return pl.pallas_call(
        kern,
        grid=(B, N // _R),
        in_specs=[xspec, xspec, xspec, xspec, wspec, wspec, wspec, wspec],
        out_specs=[
            pl.BlockSpec((1, _R, O2), lambda b, i: (b, i, 0)),
            pl.BlockSpec((1, O2), lambda b, i: (0, 0)),
            pl.BlockSpec((1, O2), lambda b, i: (0, 0)),
        ],
        out_shape=[
            jax.ShapeDtypeStruct((B, N, O2), jnp.float32),
            jax.ShapeDtypeStruct((1, O2), jnp.float32),
            jax.ShapeDtypeStruct((1, O2), jnp.float32),
        ],
    )(x1, x2, x3, x4, W1, W2, W3, W4)


def _final_bn(ypre, sc, sh):
    B, N, O = ypre.shape

    def kern(y_ref, sc_ref, sh_ref, o_ref):
        z = y_ref[0] * sc_ref[0][None, :] + sh_ref[0][None, :]
        z = jnp.where(z > 0, z, 0.2 * z)
        o_ref[0] = z.T

    return pl.pallas_call(
        kern,
        grid=(B, N // _R),
        in_specs=[
            pl.BlockSpec((1, _R, O), lambda b, i: (b, i, 0)),
            pl.BlockSpec((1, O), lambda b, i: (0, 0)),
            pl.BlockSpec((1, O), lambda b, i: (0, 0)),
        ],
        out_specs=pl.BlockSpec((1, O, _R), lambda b, i: (b, 0, i)),
        out_shape=jax.ShapeDtypeStruct((B, O, N), jnp.float32),
    )(ypre, sc, sh)


def _edge_stage(xt, Wa, ga, ba, Wb=None, gb=None, bb=None):
    B, N, C = xt.shape
    Wn = Wa[:, :C]
    Wvc = Wa[:, C:] - Wa[:, :C]
    y1, s1, ss1 = _fused_edge_gather(xt, Wn, Wvc)
    cnt = float(B * N * _K)
    sc1, sh1 = _bn_affine(s1, ss1, cnt, ga, ba)
    if Wb is None:
        return _bn_max(y1, sc1, sh1)
    y2, s2, ss2 = _edge_mm2(y1, sc1, sh1, Wb)
    sc2, sh2 = _bn_affine(s2, ss2, cnt, gb, bb)
    return _bn_max(y2, sc2, sh2)


def kernel(x, W0a, g0a, b0a, W0b, g0b, b0b, W1a, g1a, b1a, W1b, g1b, b1b,
           W2, g2, b2, W3, g3, b3, Wt, gt, bt):
    xt = jnp.transpose(x, (0, 2, 1))
    x1 = _edge_stage(xt, W0a, g0a, b0a, W0b, g0b, b0b)
    x2 = _edge_stage(x1, W1a, g1a, b1a, W1b, g1b, b1b)
    x3 = _edge_stage(x2, W2, g2, b2)
    x4 = _edge_stage(x3, W3, g3, b3)
    ypre, st, sst = _final_mm(x1, x2, x3, x4, Wt)
    B, N, _ = x1.shape
    sct, sht = _bn_affine(st, sst, float(B * N), gt, bt)
    return _final_bn(ypre, sct, sht)
